# baseline (speedup 1.0000x reference)
import functools

import jax
import jax.numpy as jnp
from jax import lax
from jax.experimental import pallas as pl
from jax.experimental.pallas import tpu as pltpu
from jax.experimental.pallas import tpu_sc as plsc

FEATURES = 32
NUM_CORES = 2
NUM_SUBCORES = 16
NUM_WORKERS = NUM_CORES * NUM_SUBCORES

K = 128
NK = 16
CHUNK = K * NK


def _embed_body(n_chunks, idx_hbm, table_hbm, out_hbm, idx_v, rows_v, sem):
    wid = lax.axis_index("s") * NUM_CORES + lax.axis_index("c")
    idx_rows_per_w = n_chunks * NK

    def chunk_body(g, carry):
        idx_row0 = wid * idx_rows_per_w + g * NK
        out_row0 = idx_row0 * K
        pltpu.sync_copy(idx_hbm.at[pl.ds(idx_row0, NK)], idx_v)
        copies = []
        for j in range(NK):
            copies.append(
                pltpu.async_copy(
                    table_hbm.at[idx_v.at[j]],
                    rows_v.at[pl.ds(j * K, K)],
                    sem,
                )
            )
        for c in copies:
            c.wait()
        pltpu.sync_copy(rows_v, out_hbm.at[pl.ds(out_row0, CHUNK)])
        return carry

    lax.fori_loop(0, n_chunks, chunk_body, 0)


def kernel(inputs, embedding):
    batch, hist = inputs.shape
    total = batch * hist
    assert total % (NUM_WORKERS * CHUNK) == 0
    n_chunks = total // (NUM_WORKERS * CHUNK)

    idx2d = inputs.reshape(total // K, K)

    num_emb = embedding.shape[0]
    feat32 = FEATURES // 2
    table_i32 = jax.lax.bitcast_convert_type(
        embedding.reshape(num_emb, feat32, 2), jnp.int32
    )

    mesh = plsc.VectorSubcoreMesh(core_axis_name="c", subcore_axis_name="s")
    run = pl.kernel(
        functools.partial(_embed_body, n_chunks),
        out_type=jax.ShapeDtypeStruct((total, feat32), jnp.int32),
        mesh=mesh,
        scratch_types=[
            pltpu.VMEM((NK, K), jnp.int32),
            pltpu.VMEM((CHUNK, feat32), jnp.int32),
            pltpu.SemaphoreType.DMA,
        ],
        compiler_params=pltpu.CompilerParams(use_tc_tiling_on_sc=False),
    )
    out = run(idx2d, table_i32)
    out_bf16 = jax.lax.bitcast_convert_type(out, jnp.bfloat16)
    return out_bf16.reshape(batch, hist, FEATURES)
